# Initial kernel scaffold; baseline (speedup 1.0000x reference)
import jax
import jax.numpy as jnp
from jax.experimental import pallas as pl


def kernel(x, edge_index, edge_attr, batch, Wrel0, brel0, Wroot0, Wrel1, brel1, Wroot1, Wrel2, brel2, Wroot2, Wrel3, brel3, Wroot3, Wrel4, brel4, Wroot4, Wrel5, brel5, Wroot5, Wrel6, brel6, Wroot6, Wl0, bl0, Wl1, bl1, Wl2, bl2, Wl3, bl3):
    raise NotImplementedError("write your pallas kernel here")


# speedup vs baseline: 1.6764x; 1.6764x over previous
import functools
import jax
import jax.numpy as jnp
from jax import lax
from jax.experimental import pallas as pl
from jax.experimental.pallas import tpu as pltpu
from jax.experimental.pallas import tpu_sc as plsc

N_NODES = 100000
N_EDGES = 1600000
NUM_GRAPHS = 64

ROW_BLK = 1000
K_EDGES = 128
PADE = 4096

_CFG = {128: (384, 128), 256: (192, 128), 512: (96, 96)}


def _layer_body(agg_ref, x_ref, wrel_ref, wroot_ref, brel_ref, out_ref):
    h = lax.dot_general(agg_ref[...], wrel_ref[...], (((1,), (1,)), ((), ())),
                        preferred_element_type=jnp.float32)
    h = h + lax.dot_general(x_ref[...], wroot_ref[...],
                            (((1,), (1,)), ((), ())),
                            preferred_element_type=jnp.float32)
    out_ref[...] = jnp.maximum(h + brel_ref[...], 0.0)


def _tc_layer(agg, x, Wrel, Wroot, brel):
    ci = x.shape[1]
    co = Wrel.shape[0]
    return pl.pallas_call(
        _layer_body,
        grid=(N_NODES // ROW_BLK,),
        in_specs=[
            pl.BlockSpec((ROW_BLK, ci), lambda i: (i, 0)),
            pl.BlockSpec((ROW_BLK, ci), lambda i: (i, 0)),
            pl.BlockSpec((co, ci), lambda i: (0, 0)),
            pl.BlockSpec((co, ci), lambda i: (0, 0)),
            pl.BlockSpec((1, co), lambda i: (0, 0)),
        ],
        out_specs=pl.BlockSpec((ROW_BLK, co), lambda i: (i, 0)),
        out_shape=jax.ShapeDtypeStruct((N_NODES, co), jnp.float32),
    )(agg, x, Wrel, Wroot, brel.reshape(1, co))


def _pre_body(pad_to, co, x_ref, wrel_ref, wroot_ref, brel_ref, y_ref,
              r_ref):
    x = x_ref[...]
    y = lax.dot_general(x, wrel_ref[...], (((1,), (1,)), ((), ())),
                        preferred_element_type=jnp.float32)
    if pad_to > co:
        y = jnp.concatenate(
            [y, jnp.zeros((ROW_BLK, pad_to - co), jnp.float32)], axis=1)
    y_ref[...] = y
    r_ref[...] = lax.dot_general(x, wroot_ref[...], (((1,), (1,)), ((), ())),
                                 preferred_element_type=jnp.float32
                                 ) + brel_ref[...]


def _tc_pre(x, Wrel, Wroot, brel, pad_to=None):
    ci = x.shape[1]
    co = Wrel.shape[0]
    pad_to = pad_to or co
    return pl.pallas_call(
        functools.partial(_pre_body, pad_to, co),
        grid=(N_NODES // ROW_BLK,),
        in_specs=[
            pl.BlockSpec((ROW_BLK, ci), lambda i: (i, 0)),
            pl.BlockSpec((co, ci), lambda i: (0, 0)),
            pl.BlockSpec((co, ci), lambda i: (0, 0)),
            pl.BlockSpec((1, co), lambda i: (0, 0)),
        ],
        out_specs=[
            pl.BlockSpec((ROW_BLK, pad_to), lambda i: (i, 0)),
            pl.BlockSpec((ROW_BLK, co), lambda i: (i, 0)),
        ],
        out_shape=[
            jax.ShapeDtypeStruct((N_NODES, pad_to), jnp.float32),
            jax.ShapeDtypeStruct((N_NODES, co), jnp.float32),
        ],
    )(x, Wrel, Wroot, brel.reshape(1, co))


def _post_body(co, a_ref, r_ref, out_ref):
    out_ref[...] = jnp.maximum(a_ref[...][:, :co] + r_ref[...], 0.0)


def _tc_post(agg, r):
    co = r.shape[1]
    return pl.pallas_call(
        functools.partial(_post_body, co),
        grid=(N_NODES // ROW_BLK,),
        in_specs=[
            pl.BlockSpec((ROW_BLK, agg.shape[1]), lambda i: (i, 0)),
            pl.BlockSpec((ROW_BLK, co), lambda i: (i, 0)),
        ],
        out_specs=pl.BlockSpec((ROW_BLK, co), lambda i: (i, 0)),
        out_shape=jax.ShapeDtypeStruct((N_NODES, co), jnp.float32),
    )(agg, r)


def _pool_body(x_ref, b_ref, sums_ref, cnts_ref):
    i = pl.program_id(0)
    b = b_ref[...]
    onehot = (b == lax.broadcasted_iota(jnp.int32, (ROW_BLK, NUM_GRAPHS), 1)
              ).astype(jnp.float32)
    s = lax.dot_general(onehot, x_ref[...], (((0,), (0,)), ((), ())),
                        preferred_element_type=jnp.float32)
    c = lax.dot_general(onehot, jnp.ones((ROW_BLK, x_ref.shape[1]),
                                         jnp.float32),
                        (((0,), (0,)), ((), ())),
                        preferred_element_type=jnp.float32)

    @pl.when(i == 0)
    def _():
        sums_ref[...] = s
        cnts_ref[...] = c

    @pl.when(i != 0)
    def _():
        sums_ref[...] += s
        cnts_ref[...] += c


def _pool(x, batch):
    return pl.pallas_call(
        _pool_body,
        grid=(N_NODES // ROW_BLK,),
        in_specs=[
            pl.BlockSpec((ROW_BLK, x.shape[1]), lambda i: (i, 0)),
            pl.BlockSpec((ROW_BLK, 1), lambda i: (i, 0)),
        ],
        out_specs=[
            pl.BlockSpec((NUM_GRAPHS, x.shape[1]), lambda i: (0, 0)),
            pl.BlockSpec((NUM_GRAPHS, x.shape[1]), lambda i: (0, 0)),
        ],
        out_shape=[
            jax.ShapeDtypeStruct((NUM_GRAPHS, x.shape[1]), jnp.float32),
            jax.ShapeDtypeStruct((NUM_GRAPHS, x.shape[1]), jnp.float32),
        ],
    )(x, batch.reshape(N_NODES, 1))


def _mlp_body(sums_ref, cnts_ref, w0, b0, w1, b1, w2, b2, w3, b3, out_ref):
    g = sums_ref[...] / jnp.maximum(cnts_ref[...], 1.0)
    for w, b, is_last in ((w0, b0, False), (w1, b1, False),
                          (w2, b2, False), (w3, b3, True)):
        g = lax.dot_general(g, w[...], (((1,), (1,)), ((), ())),
                            preferred_element_type=jnp.float32) + b[...]
        if not is_last:
            g = jnp.maximum(g, 0.0)
    out_ref[...] = g


def _mlp(sums, cnts, Wl0, bl0, Wl1, bl1, Wl2, bl2, Wl3, bl3):
    Wl3p = jnp.pad(Wl3, ((0, 127), (0, 0)))
    bl3p = jnp.pad(bl3, (0, 127))
    out = pl.pallas_call(
        _mlp_body,
        out_shape=jax.ShapeDtypeStruct((NUM_GRAPHS, 128), jnp.float32),
    )(sums, cnts,
      Wl0, bl0.reshape(1, -1), Wl1, bl1.reshape(1, -1),
      Wl2, bl2.reshape(1, -1), Wl3p, bl3p.reshape(1, -1))
    return out[:, :1]


@functools.lru_cache(maxsize=None)
def _make_agg(w):
    NPT, K = _CFG[w]
    nsb = -(-N_NODES // NPT)
    NPAD = nsb * NPT
    NB16 = ((nsb + 1 + 15) // 16) * 16
    spt = -(-nsb // 32)
    W16 = w // 16

    def body(x_ref, src_ref, dst_ref, ew_ref, bounds_ref, out_ref,
             boundsv, srcb, dstb, ewb, rows, acc, sem):
        c = lax.axis_index("c")
        s = lax.axis_index("s")
        wid = s * 2 + c
        iota = lax.iota(jnp.int32, 16)
        zero16 = jnp.zeros((16,), jnp.float32)
        pltpu.sync_copy(bounds_ref, boundsv)

        def get_b(k):
            g0 = lax.div(k, 16) * 16
            v = boundsv[pl.ds(g0, 16)]
            return jnp.sum(jnp.where(iota == (k - g0), v, 0))

        def sbloop(k0, carry):
            sb = wid + k0 * 32

            @pl.when(sb < nsb)
            def _():
                base = sb * NPT
                lo = get_b(sb)
                hi = get_b(sb + 1)
                lo8 = lax.div(lo, 8) * 8
                nbat = lax.div(hi - lo8 + K - 1, K)

                def zc(r, carry2):
                    for j in range(W16):
                        acc[r, pl.ds(j * 16, 16)] = zero16
                    return carry2
                lax.fori_loop(0, NPT + 1, zc, 0)

                def bat_body(bat, carry2):
                    estart = pl.multiple_of(lo8 + bat * K, 8)
                    pltpu.sync_copy(src_ref.at[pl.ds(estart, K)], srcb)
                    pltpu.sync_copy(dst_ref.at[pl.ds(estart, K)], dstb)
                    pltpu.sync_copy(ew_ref.at[pl.ds(estart, K)], ewb)
                    pltpu.async_copy(x_ref.at[srcb], rows, sem).wait()

                    def grp(g, carry3):
                        sl = pl.ds(g * 16, 16)
                        d = dstb[sl]
                        valid = (d >= base) & (d < base + NPT)
                        dloc = jnp.where(valid, d - base, NPT)
                        ewv = ewb[sl]
                        for l in range(16):
                            lane = iota == l
                            dl = jnp.sum(jnp.where(lane, dloc, 0))
                            ewl = jnp.sum(jnp.where(lane, ewv, 0.0))
                            ews = jnp.full((16,), ewl, jnp.float32)
                            e = g * 16 + l
                            for j in range(W16):
                                slj = pl.ds(j * 16, 16)
                                plsc.addupdate(acc.at[dl, slj],
                                               rows[e, slj] * ews)
                        return carry3
                    lax.fori_loop(0, K // 16, grp, 0)
                    return carry2
                lax.fori_loop(0, nbat, bat_body, 0)
                pltpu.sync_copy(acc.at[pl.ds(0, NPT)],
                                out_ref.at[pl.ds(base, NPT)])
            return carry
        lax.fori_loop(0, spt, sbloop, 0)

    mesh = plsc.VectorSubcoreMesh(core_axis_name="c", subcore_axis_name="s")
    return pl.kernel(
        body,
        out_type=jax.ShapeDtypeStruct((NPAD, w), jnp.float32),
        mesh=mesh,
        compiler_params=pltpu.CompilerParams(needs_layout_passes=False),
        scratch_types=[
            pltpu.VMEM((NB16,), jnp.int32),
            pltpu.VMEM((K,), jnp.int32),
            pltpu.VMEM((K,), jnp.int32),
            pltpu.VMEM((K,), jnp.float32),
            pltpu.VMEM((K, w), jnp.float32),
            pltpu.VMEM((NPT + 1, w), jnp.float32),
            pltpu.SemaphoreType.DMA,
        ],
    )


def _block_bounds(dst_s, w):
    NPT, _ = _CFG[w]
    nsb = -(-N_NODES // NPT)
    NB16 = ((nsb + 1 + 15) // 16) * 16
    starts = jnp.arange(nsb + 1, dtype=jnp.int32) * NPT
    bd = jnp.searchsorted(dst_s, starts).astype(jnp.int32)
    return jnp.pad(bd, (0, NB16 - (nsb + 1)),
                   constant_values=N_EDGES)


def kernel(x, edge_index, edge_attr, batch, Wrel0, brel0, Wroot0, Wrel1, brel1, Wroot1, Wrel2, brel2, Wroot2, Wrel3, brel3, Wroot3, Wrel4, brel4, Wroot4, Wrel5, brel5, Wroot5, Wrel6, brel6, Wroot6, Wl0, bl0, Wl1, bl1, Wl2, bl2, Wl3, bl3):
    dst_s, src_s, ew_s = lax.sort(
        (edge_index[1], edge_index[0], edge_attr), num_keys=1)
    src_p = jnp.pad(src_s, (0, PADE))
    dst_p = jnp.pad(dst_s, (0, PADE), constant_values=N_NODES)
    ew_p = jnp.pad(ew_s, (0, PADE))
    bounds = {w: _block_bounds(dst_s, w) for w in _CFG}

    def aggregate(table):
        w = table.shape[1]
        return _make_agg(w)(table, src_p, dst_p, ew_p, bounds[w])

    params = ((Wrel0, brel0, Wroot0), (Wrel1, brel1, Wroot1),
              (Wrel2, brel2, Wroot2), (Wrel3, brel3, Wroot3),
              (Wrel4, brel4, Wroot4), (Wrel5, brel5, Wroot5),
              (Wrel6, brel6, Wroot6))
    for i, (Wrel, brel, Wroot) in enumerate(params):
        if i in (0, 1, 5):
            y, r = _tc_pre(x, Wrel, Wroot, brel,
                           pad_to=128 if i == 0 else None)
            agg = aggregate(y)
            x = _tc_post(agg, r)
        else:
            agg = aggregate(x)
            x = _tc_layer(agg, x, Wrel, Wroot, brel)
    sums, cnts = _pool(x, batch)
    return _mlp(sums, cnts, Wl0, bl0, Wl1, bl1, Wl2, bl2, Wl3, bl3)


# speedup vs baseline: 1.8584x; 1.1086x over previous
import functools
import jax
import jax.numpy as jnp
from jax import lax
from jax.experimental import pallas as pl
from jax.experimental.pallas import tpu as pltpu
from jax.experimental.pallas import tpu_sc as plsc

N_NODES = 100000
N_EDGES = 1600000
NUM_GRAPHS = 64

ROW_BLK = 1000
K_EDGES = 128
PADE = 4096

_CFG = {128: (384, 192), 256: (192, 96), 512: (96, 48)}


def _layer_body(agg_ref, x_ref, wrel_ref, wroot_ref, brel_ref, out_ref):
    h = lax.dot_general(agg_ref[...], wrel_ref[...], (((1,), (1,)), ((), ())),
                        preferred_element_type=jnp.float32)
    h = h + lax.dot_general(x_ref[...], wroot_ref[...],
                            (((1,), (1,)), ((), ())),
                            preferred_element_type=jnp.float32)
    out_ref[...] = jnp.maximum(h + brel_ref[...], 0.0)


def _tc_layer(agg, x, Wrel, Wroot, brel):
    ci = x.shape[1]
    co = Wrel.shape[0]
    return pl.pallas_call(
        _layer_body,
        grid=(N_NODES // ROW_BLK,),
        in_specs=[
            pl.BlockSpec((ROW_BLK, ci), lambda i: (i, 0)),
            pl.BlockSpec((ROW_BLK, ci), lambda i: (i, 0)),
            pl.BlockSpec((co, ci), lambda i: (0, 0)),
            pl.BlockSpec((co, ci), lambda i: (0, 0)),
            pl.BlockSpec((1, co), lambda i: (0, 0)),
        ],
        out_specs=pl.BlockSpec((ROW_BLK, co), lambda i: (i, 0)),
        out_shape=jax.ShapeDtypeStruct((N_NODES, co), jnp.float32),
    )(agg, x, Wrel, Wroot, brel.reshape(1, co))


def _pre_body(pad_to, co, x_ref, wrel_ref, wroot_ref, brel_ref, y_ref,
              r_ref):
    x = x_ref[...]
    y = lax.dot_general(x, wrel_ref[...], (((1,), (1,)), ((), ())),
                        preferred_element_type=jnp.float32)
    if pad_to > co:
        y = jnp.concatenate(
            [y, jnp.zeros((ROW_BLK, pad_to - co), jnp.float32)], axis=1)
    y_ref[...] = y
    r_ref[...] = lax.dot_general(x, wroot_ref[...], (((1,), (1,)), ((), ())),
                                 preferred_element_type=jnp.float32
                                 ) + brel_ref[...]


def _tc_pre(x, Wrel, Wroot, brel, pad_to=None):
    ci = x.shape[1]
    co = Wrel.shape[0]
    pad_to = pad_to or co
    return pl.pallas_call(
        functools.partial(_pre_body, pad_to, co),
        grid=(N_NODES // ROW_BLK,),
        in_specs=[
            pl.BlockSpec((ROW_BLK, ci), lambda i: (i, 0)),
            pl.BlockSpec((co, ci), lambda i: (0, 0)),
            pl.BlockSpec((co, ci), lambda i: (0, 0)),
            pl.BlockSpec((1, co), lambda i: (0, 0)),
        ],
        out_specs=[
            pl.BlockSpec((ROW_BLK, pad_to), lambda i: (i, 0)),
            pl.BlockSpec((ROW_BLK, co), lambda i: (i, 0)),
        ],
        out_shape=[
            jax.ShapeDtypeStruct((N_NODES, pad_to), jnp.float32),
            jax.ShapeDtypeStruct((N_NODES, co), jnp.float32),
        ],
    )(x, Wrel, Wroot, brel.reshape(1, co))


def _post_body(co, a_ref, r_ref, out_ref):
    out_ref[...] = jnp.maximum(a_ref[...][:, :co] + r_ref[...], 0.0)


def _tc_post(agg, r):
    co = r.shape[1]
    return pl.pallas_call(
        functools.partial(_post_body, co),
        grid=(N_NODES // ROW_BLK,),
        in_specs=[
            pl.BlockSpec((ROW_BLK, agg.shape[1]), lambda i: (i, 0)),
            pl.BlockSpec((ROW_BLK, co), lambda i: (i, 0)),
        ],
        out_specs=pl.BlockSpec((ROW_BLK, co), lambda i: (i, 0)),
        out_shape=jax.ShapeDtypeStruct((N_NODES, co), jnp.float32),
    )(agg, r)


def _pool_body(x_ref, b_ref, sums_ref, cnts_ref):
    i = pl.program_id(0)
    b = b_ref[...]
    onehot = (b == lax.broadcasted_iota(jnp.int32, (ROW_BLK, NUM_GRAPHS), 1)
              ).astype(jnp.float32)
    s = lax.dot_general(onehot, x_ref[...], (((0,), (0,)), ((), ())),
                        preferred_element_type=jnp.float32)
    c = lax.dot_general(onehot, jnp.ones((ROW_BLK, x_ref.shape[1]),
                                         jnp.float32),
                        (((0,), (0,)), ((), ())),
                        preferred_element_type=jnp.float32)

    @pl.when(i == 0)
    def _():
        sums_ref[...] = s
        cnts_ref[...] = c

    @pl.when(i != 0)
    def _():
        sums_ref[...] += s
        cnts_ref[...] += c


def _pool(x, batch):
    return pl.pallas_call(
        _pool_body,
        grid=(N_NODES // ROW_BLK,),
        in_specs=[
            pl.BlockSpec((ROW_BLK, x.shape[1]), lambda i: (i, 0)),
            pl.BlockSpec((ROW_BLK, 1), lambda i: (i, 0)),
        ],
        out_specs=[
            pl.BlockSpec((NUM_GRAPHS, x.shape[1]), lambda i: (0, 0)),
            pl.BlockSpec((NUM_GRAPHS, x.shape[1]), lambda i: (0, 0)),
        ],
        out_shape=[
            jax.ShapeDtypeStruct((NUM_GRAPHS, x.shape[1]), jnp.float32),
            jax.ShapeDtypeStruct((NUM_GRAPHS, x.shape[1]), jnp.float32),
        ],
    )(x, batch.reshape(N_NODES, 1))


def _mlp_body(sums_ref, cnts_ref, w0, b0, w1, b1, w2, b2, w3, b3, out_ref):
    g = sums_ref[...] / jnp.maximum(cnts_ref[...], 1.0)
    for w, b, is_last in ((w0, b0, False), (w1, b1, False),
                          (w2, b2, False), (w3, b3, True)):
        g = lax.dot_general(g, w[...], (((1,), (1,)), ((), ())),
                            preferred_element_type=jnp.float32) + b[...]
        if not is_last:
            g = jnp.maximum(g, 0.0)
    out_ref[...] = g


def _mlp(sums, cnts, Wl0, bl0, Wl1, bl1, Wl2, bl2, Wl3, bl3):
    Wl3p = jnp.pad(Wl3, ((0, 127), (0, 0)))
    bl3p = jnp.pad(bl3, (0, 127))
    out = pl.pallas_call(
        _mlp_body,
        out_shape=jax.ShapeDtypeStruct((NUM_GRAPHS, 128), jnp.float32),
    )(sums, cnts,
      Wl0, bl0.reshape(1, -1), Wl1, bl1.reshape(1, -1),
      Wl2, bl2.reshape(1, -1), Wl3p, bl3p.reshape(1, -1))
    return out[:, :1]


@functools.lru_cache(maxsize=None)
def _make_agg(w):
    NPT, K = _CFG[w]
    nsb = -(-N_NODES // NPT)
    NPAD = nsb * NPT
    NB16 = ((nsb + 1 + 15) // 16) * 16
    spt = -(-nsb // 32)
    W16 = w // 16

    def body(x_ref, src_ref, dst_ref, ew_ref, bounds_ref, out_ref,
             boundsv, src2, dst2, ew2, rows2, acc, semA, semB):
        c = lax.axis_index("c")
        s = lax.axis_index("s")
        wid = s * 2 + c
        iota = lax.iota(jnp.int32, 16)
        zero16 = jnp.zeros((16,), jnp.float32)
        pltpu.sync_copy(bounds_ref, boundsv)

        def get_b(k):
            g0 = lax.div(k, 16) * 16
            v = boundsv[pl.ds(g0, 16)]
            return jnp.sum(jnp.where(iota == (k - g0), v, 0))

        def sbloop(k0, carry):
            sb = wid + k0 * 32

            @pl.when(sb < nsb)
            def _():
                base = sb * NPT
                lo = get_b(sb)
                hi = get_b(sb + 1)
                lo8 = lax.div(lo, 8) * 8
                nbat = lax.div(hi - lo8 + K - 1, K)

                def zc(r, carry2):
                    for j in range(W16):
                        acc[r, pl.ds(j * 16, 16)] = zero16
                    return carry2
                lax.fori_loop(0, NPT + 1, zc, 0)

                def start(bat):
                    slot = lax.rem(bat, 2)
                    off = slot * K
                    estart = pl.multiple_of(lo8 + bat * K, 8)
                    esl = pl.ds(estart, K)
                    osl = pl.ds(off, K)
                    pltpu.sync_copy(src_ref.at[esl], src2.at[osl])
                    pltpu.sync_copy(dst_ref.at[esl], dst2.at[osl])
                    pltpu.sync_copy(ew_ref.at[esl], ew2.at[osl])

                    @pl.when(slot == 0)
                    def _():
                        pltpu.async_copy(x_ref.at[src2.at[pl.ds(0, K)]],
                                         rows2.at[pl.ds(0, K)], semA)

                    @pl.when(slot == 1)
                    def _():
                        pltpu.async_copy(x_ref.at[src2.at[pl.ds(K, K)]],
                                         rows2.at[pl.ds(K, K)], semB)

                @pl.when(nbat > 0)
                def _():
                    start(0)

                def bat_body(bat, carry2):
                    @pl.when(bat + 1 < nbat)
                    def _():
                        start(bat + 1)
                    slot = lax.rem(bat, 2)
                    off = slot * K

                    @pl.when(slot == 0)
                    def _():
                        pltpu.make_async_copy(
                            x_ref.at[src2.at[pl.ds(0, K)]],
                            rows2.at[pl.ds(0, K)], semA).wait()

                    @pl.when(slot == 1)
                    def _():
                        pltpu.make_async_copy(
                            x_ref.at[src2.at[pl.ds(K, K)]],
                            rows2.at[pl.ds(K, K)], semB).wait()

                    def grp(g, carry3):
                        sl = pl.ds(off + g * 16, 16)
                        d = dst2[sl]
                        valid = (d >= base) & (d < base + NPT)
                        dloc = jnp.where(valid, d - base, NPT)
                        ewv = ew2[sl]
                        for l in range(16):
                            lane = iota == l
                            dl = jnp.sum(jnp.where(lane, dloc, 0))
                            ewl = jnp.sum(jnp.where(lane, ewv, 0.0))
                            ews = jnp.full((16,), ewl, jnp.float32)
                            e = off + g * 16 + l
                            for j in range(W16):
                                slj = pl.ds(j * 16, 16)
                                plsc.addupdate(acc.at[dl, slj],
                                               rows2[e, slj] * ews)
                        return carry3
                    lax.fori_loop(0, K // 16, grp, 0)
                    return carry2
                lax.fori_loop(0, nbat, bat_body, 0)
                pltpu.sync_copy(acc.at[pl.ds(0, NPT)],
                                out_ref.at[pl.ds(base, NPT)])
            return carry
        lax.fori_loop(0, spt, sbloop, 0)

    mesh = plsc.VectorSubcoreMesh(core_axis_name="c", subcore_axis_name="s")
    return pl.kernel(
        body,
        out_type=jax.ShapeDtypeStruct((NPAD, w), jnp.float32),
        mesh=mesh,
        compiler_params=pltpu.CompilerParams(needs_layout_passes=False),
        scratch_types=[
            pltpu.VMEM((NB16,), jnp.int32),
            pltpu.VMEM((2 * K,), jnp.int32),
            pltpu.VMEM((2 * K,), jnp.int32),
            pltpu.VMEM((2 * K,), jnp.float32),
            pltpu.VMEM((2 * K, w), jnp.float32),
            pltpu.VMEM((NPT + 1, w), jnp.float32),
            pltpu.SemaphoreType.DMA,
            pltpu.SemaphoreType.DMA,
        ],
    )


def _block_bounds(dst_s, w):
    NPT, _ = _CFG[w]
    nsb = -(-N_NODES // NPT)
    NB16 = ((nsb + 1 + 15) // 16) * 16
    starts = jnp.arange(nsb + 1, dtype=jnp.int32) * NPT
    bd = jnp.searchsorted(dst_s, starts).astype(jnp.int32)
    return jnp.pad(bd, (0, NB16 - (nsb + 1)),
                   constant_values=N_EDGES)


def kernel(x, edge_index, edge_attr, batch, Wrel0, brel0, Wroot0, Wrel1, brel1, Wroot1, Wrel2, brel2, Wroot2, Wrel3, brel3, Wroot3, Wrel4, brel4, Wroot4, Wrel5, brel5, Wroot5, Wrel6, brel6, Wroot6, Wl0, bl0, Wl1, bl1, Wl2, bl2, Wl3, bl3):
    dst_s, src_s, ew_s = lax.sort(
        (edge_index[1], edge_index[0], edge_attr), num_keys=1)
    src_p = jnp.pad(src_s, (0, PADE))
    dst_p = jnp.pad(dst_s, (0, PADE), constant_values=N_NODES)
    ew_p = jnp.pad(ew_s, (0, PADE))
    bounds = {w: _block_bounds(dst_s, w) for w in _CFG}

    def aggregate(table):
        w = table.shape[1]
        return _make_agg(w)(table, src_p, dst_p, ew_p, bounds[w])

    params = ((Wrel0, brel0, Wroot0), (Wrel1, brel1, Wroot1),
              (Wrel2, brel2, Wroot2), (Wrel3, brel3, Wroot3),
              (Wrel4, brel4, Wroot4), (Wrel5, brel5, Wroot5),
              (Wrel6, brel6, Wroot6))
    for i, (Wrel, brel, Wroot) in enumerate(params):
        if i in (0, 1, 5):
            y, r = _tc_pre(x, Wrel, Wroot, brel,
                           pad_to=128 if i == 0 else None)
            agg = aggregate(y)
            x = _tc_post(agg, r)
        else:
            agg = aggregate(x)
            x = _tc_layer(agg, x, Wrel, Wroot, brel)
    sums, cnts = _pool(x, batch)
    return _mlp(sums, cnts, Wl0, bl0, Wl1, bl1, Wl2, bl2, Wl3, bl3)
